# baseline (speedup 1.0000x reference)
import numpy as np
import jax
import jax.numpy as jnp
from jax.experimental import pallas as pl
from jax.experimental.pallas import tpu as pltpu

_BN_EPS = 1e-5

_B, _C, _H, _W = 16, 256, 64, 64
_T = _H * _W
_TA = _T // 4
_CHUNK = 512
_NCHUNK = _TA // _CHUNK
_OUT_C = 512


def _spa_full_idx() -> np.ndarray:
    idx = np.arange(_T)
    a_idx = idx[::4]
    b_idx = idx[idx % 4 != 0]
    width = int(np.sqrt(_T))
    ac = np.stack([a_idx // width, a_idx % width], -1).astype(np.float32)
    bc = np.stack([b_idx // width, b_idx % width], -1).astype(np.float32)
    dist = np.sqrt(((ac[:, None, :] - bc[None, :, :]) ** 2).sum(-1))
    return b_idx[np.argmax(1.0 / (dist + 1e-6), axis=-1)]


_IDX2 = np.concatenate([np.arange(0, _T, 4), _spa_full_idx()])


def _body(x_ref, xas_ref, w_ref, g_ref, be_ref, mu_ref, va_ref,
          fw_ref, o_ref):
    xf = x_ref[0]
    xas = xas_ref[0]

    bn = xf / jnp.sqrt(jnp.sum(xf * xf, axis=0, keepdims=True))

    fw = jnp.clip(fw_ref[...], 0.0, 6.0)
    fwn = fw / (jnp.sum(fw) + 1e-8)
    csim = 0.5 * fwn[0, 0]
    cspa = 0.5 * fwn[0, 1]

    scale = g_ref[...] / jnp.sqrt(va_ref[...] + _BN_EPS)
    bias = be_ref[...] - mu_ref[...] * scale

    r8 = jax.lax.broadcasted_iota(jnp.int32, (8, 128), 0)
    pen8 = jnp.where((r8 & 3) == 0, -3e38, 0.0).astype(jnp.float32)
    pen = pltpu.repeat(pltpu.repeat(pen8, _T // 8, 0), _CHUNK // 128, 1)

    for c in range(_NCHUNK):
        lo = c * _CHUNK
        xa = xas[:, lo:lo + _CHUNK]
        xspa = xas[:, _TA + lo:_TA + lo + _CHUNK]

        an = xa / jnp.sqrt(jnp.sum(xa * xa, axis=0, keepdims=True))

        simT = jax.lax.dot_general(bn, an, (((0,), (0,)), ((), ())),
                                   preferred_element_type=jnp.float32)
        simT = simT + pen

        m = jnp.max(simT, axis=0, keepdims=True)
        masked = jnp.where(simT == m, simT, 0.0)

        selp = jax.lax.dot_general(xf, masked, (((1,), (0,)), ((), ())),
                                   preferred_element_type=jnp.float32)
        sel = selp * (1.0 / m)
        fused = (csim + cspa) * xa + cspa * xspa + csim * sel

        out = jax.lax.dot_general(w_ref[...], fused, (((1,), (0,)), ((), ())),
                                  preferred_element_type=jnp.float32)
        y = out * scale + bias
        o_ref[0, :, lo:lo + _CHUNK] = y * jax.nn.sigmoid(y)


def kernel(x, conv_w, bn_gamma, bn_beta, bn_mean, bn_var, fusion_weights):
    B, C, H, W = x.shape
    xr = x.reshape(B, C, _T)
    x4 = x.reshape(B, C, _TA, 4)
    xa_full = x4[..., 0]
    p1 = x4[..., 1]
    p3 = x4[..., 3]
    xm = jnp.concatenate([p3[:, :, :1], p3[:, :, :-1]], axis=2)
    first_col = jnp.asarray((np.arange(_TA) % 16 == 0)[None, None, :])
    xspa_full = jnp.where(first_col, p1, xm)
    xas = jnp.concatenate([xa_full, xspa_full], axis=2)

    grid = (2, B // 2)
    bat = lambda p, j: p * (B // 2) + j
    out = pl.pallas_call(
        _body,
        grid=grid,
        in_specs=[
            pl.BlockSpec((1, C, _T), lambda p, j: (bat(p, j), 0, 0)),
            pl.BlockSpec((1, C, 2 * _TA), lambda p, j: (bat(p, j), 0, 0)),
            pl.BlockSpec((_OUT_C, C), lambda p, j: (0, 0)),
            pl.BlockSpec((_OUT_C, 1), lambda p, j: (0, 0)),
            pl.BlockSpec((_OUT_C, 1), lambda p, j: (0, 0)),
            pl.BlockSpec((_OUT_C, 1), lambda p, j: (0, 0)),
            pl.BlockSpec((_OUT_C, 1), lambda p, j: (0, 0)),
            pl.BlockSpec((1, 2), lambda p, j: (0, 0)),
        ],
        out_specs=pl.BlockSpec((1, _OUT_C, _TA),
                               lambda p, j: (bat(p, j), 0, 0)),
        out_shape=jax.ShapeDtypeStruct((B, _OUT_C, _TA), jnp.float32),
        compiler_params=pltpu.CompilerParams(
            dimension_semantics=("parallel", "arbitrary"),
            vmem_limit_bytes=100 * 1024 * 1024,
        ),
    )(
        xr, xas, conv_w,
        bn_gamma.reshape(_OUT_C, 1), bn_beta.reshape(_OUT_C, 1),
        bn_mean.reshape(_OUT_C, 1), bn_var.reshape(_OUT_C, 1),
        fusion_weights.reshape(1, 2),
    )
    return out.reshape(B, _OUT_C, H // 2, W // 2)


# speedup vs baseline: 1.2055x; 1.2055x over previous
import numpy as np
import jax
import jax.numpy as jnp
from jax.experimental import pallas as pl
from jax.experimental.pallas import tpu as pltpu

_BN_EPS = 1e-5

_B, _C, _H, _W = 16, 256, 64, 64
_T = _H * _W
_TA = _T // 4
_CHUNK = 512
_NCHUNK = _TA // _CHUNK
_OUT_C = 512


def _spa_full_idx() -> np.ndarray:
    idx = np.arange(_T)
    a_idx = idx[::4]
    b_idx = idx[idx % 4 != 0]
    width = int(np.sqrt(_T))
    ac = np.stack([a_idx // width, a_idx % width], -1).astype(np.float32)
    bc = np.stack([b_idx // width, b_idx % width], -1).astype(np.float32)
    dist = np.sqrt(((ac[:, None, :] - bc[None, :, :]) ** 2).sum(-1))
    return b_idx[np.argmax(1.0 / (dist + 1e-6), axis=-1)]


_IDX2 = np.concatenate([np.arange(0, _T, 4), _spa_full_idx()])


def _body(x_ref, xas_ref, w_ref, g_ref, be_ref, mu_ref, va_ref,
          fw_ref, o_ref):
    xf = x_ref[0]
    xas = xas_ref[0]

    bn = xf / jnp.sqrt(jnp.sum(xf * xf, axis=0, keepdims=True))

    fw = jnp.clip(fw_ref[...], 0.0, 6.0)
    fwn = fw / (jnp.sum(fw) + 1e-8)
    csim = 0.5 * fwn[0, 0]
    cspa = 0.5 * fwn[0, 1]

    scale = g_ref[...] / jnp.sqrt(va_ref[...] + _BN_EPS)
    bias = be_ref[...] - mu_ref[...] * scale

    r8 = jax.lax.broadcasted_iota(jnp.int32, (8, 128), 0)
    pen8 = jnp.where((r8 & 3) == 0, -3e38, 0.0).astype(jnp.float32)
    pen = pltpu.repeat(pltpu.repeat(pen8, _T // 8, 0), _CHUNK // 128, 1)

    for c in range(_NCHUNK):
        lo = c * _CHUNK
        xa = xas[:, lo:lo + _CHUNK]
        xspa = xas[:, _TA + lo:_TA + lo + _CHUNK]

        an = xa / jnp.sqrt(jnp.sum(xa * xa, axis=0, keepdims=True))

        simT = jax.lax.dot_general(bn, an, (((0,), (0,)), ((), ())),
                                   preferred_element_type=jnp.float32)
        simT = simT + pen

        m = jnp.max(simT, axis=0, keepdims=True)
        masked = jnp.where(simT == m, simT, 0.0)

        selp = jax.lax.dot_general(xf, masked, (((1,), (0,)), ((), ())),
                                   preferred_element_type=jnp.float32)
        sel = selp * (1.0 / m)
        fused = (csim + cspa) * xa + cspa * xspa + csim * sel

        out = jax.lax.dot_general(w_ref[...], fused, (((1,), (0,)), ((), ())),
                                  preferred_element_type=jnp.float32)
        y = out * scale + bias
        o_ref[0, :, lo:lo + _CHUNK] = y * jax.nn.sigmoid(y)


def kernel(x, conv_w, bn_gamma, bn_beta, bn_mean, bn_var, fusion_weights):
    B, C, H, W = x.shape
    xr = x.reshape(B, C, _T)
    xas = x[:, :, _IDX2 // _W, _IDX2 % _W]

    grid = (2, B // 2)
    bat = lambda p, j: p * (B // 2) + j
    out = pl.pallas_call(
        _body,
        grid=grid,
        in_specs=[
            pl.BlockSpec((1, C, _T), lambda p, j: (bat(p, j), 0, 0)),
            pl.BlockSpec((1, C, 2 * _TA), lambda p, j: (bat(p, j), 0, 0)),
            pl.BlockSpec((_OUT_C, C), lambda p, j: (0, 0)),
            pl.BlockSpec((_OUT_C, 1), lambda p, j: (0, 0)),
            pl.BlockSpec((_OUT_C, 1), lambda p, j: (0, 0)),
            pl.BlockSpec((_OUT_C, 1), lambda p, j: (0, 0)),
            pl.BlockSpec((_OUT_C, 1), lambda p, j: (0, 0)),
            pl.BlockSpec((1, 2), lambda p, j: (0, 0)),
        ],
        out_specs=pl.BlockSpec((1, _OUT_C, _TA),
                               lambda p, j: (bat(p, j), 0, 0)),
        out_shape=jax.ShapeDtypeStruct((B, _OUT_C, _TA), jnp.float32),
        compiler_params=pltpu.CompilerParams(
            dimension_semantics=("parallel", "arbitrary"),
            vmem_limit_bytes=100 * 1024 * 1024,
        ),
    )(
        xr, xas, conv_w,
        bn_gamma.reshape(_OUT_C, 1), bn_beta.reshape(_OUT_C, 1),
        bn_mean.reshape(_OUT_C, 1), bn_var.reshape(_OUT_C, 1),
        fusion_weights.reshape(1, 2),
    )
    return out.reshape(B, _OUT_C, H // 2, W // 2)
